# baseline (speedup 1.0000x reference)
import jax
import jax.numpy as jnp
from jax import lax
from jax.experimental import pallas as pl
from jax.experimental.pallas import tpu as pltpu
from jax.experimental.pallas import tpu_sc as plsc

LANES = 16
NW = 32
BW = 512
LC = 5


def _sc_body(B, L, idx_hbm, tab_hbm, lab_hbm, coded_hbm, mask_hbm,
             labo_hbm, idx_blk, coded_s, tab_v, lab_v, sem_c, sem_m):
    c = lax.axis_index("c")
    s = lax.axis_index("s")
    wid = s * 2 + c

    pltpu.sync_copy(lab_hbm.at[pl.ds(wid * BW, BW)], lab_v)
    pltpu.sync_copy(lab_v, labo_hbm.at[pl.ds(wid * BW, BW)])

    pltpu.sync_copy(tab_hbm, tab_v.at[pl.ds(LANES, LANES)])

    lanes = lax.iota(jnp.int32, LANES)
    lanesL = lanes * L

    b0 = wid * BW
    pltpu.sync_copy(idx_hbm.at[pl.ds(b0 * L, BW * L)], idx_blk)
    mid0 = wid * (BW // 128) * 4

    def chunk_body(ci, carry):
        p = ci & 1
        l0 = ci * LC
        cbuf = coded_s.at[p]

        @pl.when(ci > 1)
        def _():
            pltpu.make_async_copy(
                cbuf, coded_hbm.at[pl.ds(0, LC), pl.ds(mid0, 16)],
                sem_c.at[p]).wait()
            pltpu.make_async_copy(
                cbuf, mask_hbm.at[pl.ds(0, LC), pl.ds(mid0, 16)],
                sem_m.at[p]).wait()

        for l_i in range(LC):
            l_abs = l0 + l_i
            for bt2 in range(BW // 128):
                gis = []
                for g in range(128 // LANES):
                    bidx = (bt2 * 128 + g * LANES) * L + l_abs + lanesL
                    gis.append(plsc.load_gather(idx_blk, [bidx]) << 2)
                for ch in range(4):
                    for g in range(128 // LANES):
                        vals = plsc.load_gather(
                            tab_v, [gis[g] + (LANES + ch)])
                        cbuf[l_i, bt2 * 4 + ch,
                             pl.ds(g * LANES, LANES)] = vals

        pltpu.async_copy(
            cbuf, coded_hbm.at[pl.ds(l0, LC), pl.ds(mid0, 16)], sem_c.at[p])
        pltpu.async_copy(
            cbuf, mask_hbm.at[pl.ds(l0, LC), pl.ds(mid0, 16)], sem_m.at[p])
        return carry
    lax.fori_loop(0, L // LC, chunk_body, 0)

    for p in range(2):
        cbuf = coded_s.at[p]
        pltpu.make_async_copy(
            cbuf, coded_hbm.at[pl.ds(0, LC), pl.ds(mid0, 16)],
            sem_c.at[p]).wait()
        pltpu.make_async_copy(
            cbuf, mask_hbm.at[pl.ds(0, LC), pl.ds(mid0, 16)],
            sem_m.at[p]).wait()


def _tc_embed_body(tab_ref, out_ref, pat_ref):
    @pl.when(pl.program_id(0) == 0)
    def _():
        mid = lax.broadcasted_iota(jnp.int32, (2048, 128), 0)
        e = (mid // 1024) * 8 + (mid % 8)
        acc = jnp.zeros((2048, 128), jnp.float32)
        for i in range(16):
            acc = jnp.where(e == i, tab_ref[i], acc)
        pat_ref[...] = acc
    out_ref[...] = pat_ref[...][None]


def kernel(indices, labels, table):
    B, L = indices.shape
    assert B == NW * BW
    assert L % (2 * LC) == 0
    BT = B // 128

    idx_flat = indices.astype(jnp.int32).reshape(B * L)
    tab_flat = table.reshape(16).astype(jnp.float32)

    mesh = plsc.VectorSubcoreMesh(core_axis_name="c", subcore_axis_name="s")
    body = lambda *args: _sc_body(B, L, *args)
    coded_x, mask_x, labels_o = pl.kernel(
        body,
        out_type=[
            jax.ShapeDtypeStruct((L, BT * 4, 128), jnp.float32),
            jax.ShapeDtypeStruct((L, BT * 4, 128), jnp.float32),
            jax.ShapeDtypeStruct((B,), jnp.float32),
        ],
        mesh=mesh,
        compiler_params=pltpu.CompilerParams(needs_layout_passes=False),
        scratch_types=[
            pltpu.VMEM((BW * L,), jnp.int32),
            pltpu.VMEM((2, LC, 16, 128), jnp.float32),
            pltpu.VMEM((2 * LANES,), jnp.float32),
            pltpu.VMEM((BW,), jnp.float32),
            pltpu.SemaphoreType.DMA((2,)),
            pltpu.SemaphoreType.DMA((2,)),
        ],
    )(idx_flat, tab_flat, labels)

    embed_x = pl.pallas_call(
        _tc_embed_body,
        grid=(L,),
        in_specs=[pl.BlockSpec(memory_space=pltpu.SMEM)],
        out_specs=pl.BlockSpec((1, BT * 16, 128), lambda i: (i, 0, 0)),
        out_shape=jax.ShapeDtypeStruct((L, BT * 16, 128), jnp.float32),
        scratch_shapes=[pltpu.VMEM((BT * 16, 128), jnp.float32)],
    )(tab_flat)

    coded = (coded_x.reshape(L, BT, 4, 128).transpose(1, 3, 0, 2)
             .reshape(B, L, 4))
    embed = (embed_x.reshape(L, 2, BT, 8, 128).transpose(2, 4, 0, 1, 3)
             .reshape(B, L, 16))
    mask = (mask_x.reshape(L, BT, 4, 128).transpose(1, 3, 0, 2)
            .reshape(B, L, 4))
    return coded, embed, mask, labels_o


# speedup vs baseline: 1.1941x; 1.1941x over previous
import jax
import jax.numpy as jnp
from jax import lax
from jax.experimental import pallas as pl
from jax.experimental.pallas import tpu as pltpu
from jax.experimental.pallas import tpu_sc as plsc

LANES = 16
NW = 32
BW = 512
LC = 5


def _sc_body(B, L, idx_hbm, tab_hbm, lab_hbm, coded_hbm, mask_hbm,
             labo_hbm, idx_blk, coded_s, tab_v, lab_v, sem_c, sem_m):
    c = lax.axis_index("c")
    s = lax.axis_index("s")
    wid = s * 2 + c

    pltpu.sync_copy(lab_hbm.at[pl.ds(wid * BW, BW)], lab_v)
    pltpu.sync_copy(lab_v, labo_hbm.at[pl.ds(wid * BW, BW)])

    pltpu.sync_copy(tab_hbm, tab_v.at[pl.ds(LANES, LANES)])

    lanes = lax.iota(jnp.int32, LANES)
    lanesL = lanes * L

    b0 = wid * BW
    pltpu.sync_copy(idx_hbm.at[pl.ds(b0 * L, BW * L)], idx_blk)
    mid0 = wid * (BW // 128) * 4

    def chunk_body(ci, carry):
        p = ci & 1
        l0 = ci * LC
        cbuf = coded_s.at[p]

        @pl.when(ci > 1)
        def _():
            pltpu.make_async_copy(
                cbuf, coded_hbm.at[pl.ds(0, LC), pl.ds(mid0, 16)],
                sem_c.at[p]).wait()
            pltpu.make_async_copy(
                cbuf, mask_hbm.at[pl.ds(0, LC), pl.ds(mid0, 16)],
                sem_m.at[p]).wait()

        @plsc.parallel_loop(0, LC * (BW // 128) * (128 // LANES), unroll=8)
        def _(i):
            l_i = i >> 5
            bt2 = (i >> 3) & 3
            g = i & 7
            bidx = (bt2 * 128 + g * LANES) * L + (l0 + l_i) + lanesL
            gi4 = plsc.load_gather(idx_blk, [bidx]) << 2
            for ch in range(4):
                vals = plsc.load_gather(tab_v, [gi4 + (LANES + ch)])
                cbuf[l_i, bt2 * 4 + ch, pl.ds(g * LANES, LANES)] = vals

        pltpu.async_copy(
            cbuf, coded_hbm.at[pl.ds(l0, LC), pl.ds(mid0, 16)], sem_c.at[p])
        pltpu.async_copy(
            cbuf, mask_hbm.at[pl.ds(l0, LC), pl.ds(mid0, 16)], sem_m.at[p])
        return carry
    lax.fori_loop(0, L // LC, chunk_body, 0)

    for p in range(2):
        cbuf = coded_s.at[p]
        pltpu.make_async_copy(
            cbuf, coded_hbm.at[pl.ds(0, LC), pl.ds(mid0, 16)],
            sem_c.at[p]).wait()
        pltpu.make_async_copy(
            cbuf, mask_hbm.at[pl.ds(0, LC), pl.ds(mid0, 16)],
            sem_m.at[p]).wait()


def _tc_embed_body(tab_ref, out_ref, pat_ref):
    @pl.when(pl.program_id(0) == 0)
    def _():
        mid = lax.broadcasted_iota(jnp.int32, (2048, 128), 0)
        e = (mid // 1024) * 8 + (mid % 8)
        acc = jnp.zeros((2048, 128), jnp.float32)
        for i in range(16):
            acc = jnp.where(e == i, tab_ref[i], acc)
        pat_ref[...] = acc
    out_ref[...] = pat_ref[...][None]


def kernel(indices, labels, table):
    B, L = indices.shape
    assert B == NW * BW
    assert L % (2 * LC) == 0
    BT = B // 128

    idx_flat = indices.astype(jnp.int32).reshape(B * L)
    tab_flat = table.reshape(16).astype(jnp.float32)

    mesh = plsc.VectorSubcoreMesh(core_axis_name="c", subcore_axis_name="s")
    body = lambda *args: _sc_body(B, L, *args)
    coded_x, mask_x, labels_o = pl.kernel(
        body,
        out_type=[
            jax.ShapeDtypeStruct((L, BT * 4, 128), jnp.float32),
            jax.ShapeDtypeStruct((L, BT * 4, 128), jnp.float32),
            jax.ShapeDtypeStruct((B,), jnp.float32),
        ],
        mesh=mesh,
        compiler_params=pltpu.CompilerParams(needs_layout_passes=False),
        scratch_types=[
            pltpu.VMEM((BW * L,), jnp.int32),
            pltpu.VMEM((2, LC, 16, 128), jnp.float32),
            pltpu.VMEM((2 * LANES,), jnp.float32),
            pltpu.VMEM((BW,), jnp.float32),
            pltpu.SemaphoreType.DMA((2,)),
            pltpu.SemaphoreType.DMA((2,)),
        ],
    )(idx_flat, tab_flat, labels)

    embed_x = pl.pallas_call(
        _tc_embed_body,
        grid=(L,),
        in_specs=[pl.BlockSpec(memory_space=pltpu.SMEM)],
        out_specs=pl.BlockSpec((1, BT * 16, 128), lambda i: (i, 0, 0)),
        out_shape=jax.ShapeDtypeStruct((L, BT * 16, 128), jnp.float32),
        scratch_shapes=[pltpu.VMEM((BT * 16, 128), jnp.float32)],
    )(tab_flat)

    coded = (coded_x.reshape(L, BT, 4, 128).transpose(1, 3, 0, 2)
             .reshape(B, L, 4))
    embed = (embed_x.reshape(L, 2, BT, 8, 128).transpose(2, 4, 0, 1, 3)
             .reshape(B, L, 16))
    mask = (mask_x.reshape(L, BT, 4, 128).transpose(1, 3, 0, 2)
            .reshape(B, L, 4))
    return coded, embed, mask, labels_o


# speedup vs baseline: 1.4325x; 1.1996x over previous
import jax
import jax.numpy as jnp
from jax import lax
from jax.experimental import pallas as pl
from jax.experimental.pallas import tpu as pltpu
from jax.experimental.pallas import tpu_sc as plsc

LANES = 16
NW = 32
BW = 512
LC = 5


def _sc_body(B, L, idx_hbm, tab_hbm, lab_hbm, coded_hbm, mask_hbm,
             labo_hbm, idx_blk, coded_s, tab_v, lab_v, sem_c, sem_m):
    c = lax.axis_index("c")
    s = lax.axis_index("s")
    wid = s * 2 + c

    pltpu.sync_copy(lab_hbm.at[pl.ds(wid * BW, BW)], lab_v)
    pltpu.sync_copy(lab_v, labo_hbm.at[pl.ds(wid * BW, BW)])

    pltpu.sync_copy(tab_hbm, tab_v.at[pl.ds(LANES, LANES)])

    lanes = lax.iota(jnp.int32, LANES)
    lanesL = lanes * L

    b0 = wid * BW
    pltpu.sync_copy(idx_hbm.at[pl.ds(b0 * L, BW * L)], idx_blk)
    mid0 = wid * (BW // 128) * 4

    def chunk_body(ci, carry):
        p = ci & 1
        l0 = ci * LC
        cbuf = coded_s.at[p]

        @pl.when(ci > 1)
        def _():
            pltpu.make_async_copy(
                cbuf, coded_hbm.at[pl.ds(0, LC), pl.ds(mid0, 16)],
                sem_c.at[p]).wait()
            pltpu.make_async_copy(
                cbuf, mask_hbm.at[pl.ds(0, LC), pl.ds(mid0, 16)],
                sem_m.at[p]).wait()

        @plsc.parallel_loop(0, LC * (BW // 128) * (128 // LANES), unroll=8)
        def _(i):
            l_i = i >> 5
            bt2 = (i >> 3) & 3
            g = i & 7
            bidx = (bt2 * 128 + g * LANES) * L + (l0 + l_i) + lanesL
            gi4 = plsc.load_gather(idx_blk, [bidx]) << 2
            for ch in range(4):
                vals = plsc.load_gather(tab_v, [gi4 + (LANES + ch)])
                cbuf[l_i, bt2 * 4 + ch, pl.ds(g * LANES, LANES)] = vals

        pltpu.async_copy(
            cbuf, coded_hbm.at[pl.ds(l0, LC), pl.ds(mid0, 16)], sem_c.at[p])
        pltpu.async_copy(
            cbuf, mask_hbm.at[pl.ds(l0, LC), pl.ds(mid0, 16)], sem_m.at[p])
        return carry
    lax.fori_loop(0, L // LC, chunk_body, 0)

    for p in range(2):
        cbuf = coded_s.at[p]
        pltpu.make_async_copy(
            cbuf, coded_hbm.at[pl.ds(0, LC), pl.ds(mid0, 16)],
            sem_c.at[p]).wait()
        pltpu.make_async_copy(
            cbuf, mask_hbm.at[pl.ds(0, LC), pl.ds(mid0, 16)],
            sem_m.at[p]).wait()


def _tc_embed_body(tab_ref, out_ref, pat_ref):
    @pl.when(pl.program_id(0) == 0)
    def _():
        mid = lax.broadcasted_iota(jnp.int32, (2048, 128), 0)
        e = (mid // 1024) * 8 + (mid % 8)
        acc = jnp.zeros((2048, 128), jnp.float32)
        for i in range(16):
            acc = jnp.where(e == i, tab_ref[i], acc)
        pat_ref[...] = acc
    pat = pat_ref[...]
    for r in range(8):
        out_ref[r] = pat


def kernel(indices, labels, table):
    B, L = indices.shape
    assert B == NW * BW
    assert L % (2 * LC) == 0
    BT = B // 128

    idx_flat = indices.astype(jnp.int32).reshape(B * L)
    tab_flat = table.reshape(16).astype(jnp.float32)

    mesh = plsc.VectorSubcoreMesh(core_axis_name="c", subcore_axis_name="s")
    body = lambda *args: _sc_body(B, L, *args)
    coded_x, mask_x, labels_o = pl.kernel(
        body,
        out_type=[
            jax.ShapeDtypeStruct((L, BT * 4, 128), jnp.float32),
            jax.ShapeDtypeStruct((L, BT * 4, 128), jnp.float32),
            jax.ShapeDtypeStruct((B,), jnp.float32),
        ],
        mesh=mesh,
        compiler_params=pltpu.CompilerParams(needs_layout_passes=False),
        scratch_types=[
            pltpu.VMEM((BW * L,), jnp.int32),
            pltpu.VMEM((2, LC, 16, 128), jnp.float32),
            pltpu.VMEM((2 * LANES,), jnp.float32),
            pltpu.VMEM((BW,), jnp.float32),
            pltpu.SemaphoreType.DMA((2,)),
            pltpu.SemaphoreType.DMA((2,)),
        ],
    )(idx_flat, tab_flat, labels)

    embed_x = pl.pallas_call(
        _tc_embed_body,
        grid=(L // 8,),
        in_specs=[pl.BlockSpec(memory_space=pltpu.SMEM)],
        out_specs=pl.BlockSpec((8, BT * 16, 128), lambda i: (i, 0, 0)),
        out_shape=jax.ShapeDtypeStruct((L, BT * 16, 128), jnp.float32),
        scratch_shapes=[pltpu.VMEM((BT * 16, 128), jnp.float32)],
    )(tab_flat)

    coded = (coded_x.reshape(L, BT, 4, 128).transpose(1, 3, 0, 2)
             .reshape(B, L, 4))
    embed = (embed_x.reshape(L, 2, BT, 8, 128).transpose(2, 4, 0, 1, 3)
             .reshape(B, L, 16))
    mask = (mask_x.reshape(L, BT, 4, 128).transpose(1, 3, 0, 2)
            .reshape(B, L, 4))
    return coded, embed, mask, labels_o


# speedup vs baseline: 1.4346x; 1.0015x over previous
import jax
import jax.numpy as jnp
from jax import lax
from jax.experimental import pallas as pl
from jax.experimental.pallas import tpu as pltpu
from jax.experimental.pallas import tpu_sc as plsc

LANES = 16
NW = 32
BW = 512
LC = 5


def _sc_body(B, L, idx_hbm, tab_hbm, lab_hbm, coded_hbm, mask_hbm,
             labo_hbm, idx_blk, coded_s, tab_v, lab_v, sem_c, sem_m):
    c = lax.axis_index("c")
    s = lax.axis_index("s")
    wid = s * 2 + c

    pltpu.sync_copy(lab_hbm.at[pl.ds(wid * BW, BW)], lab_v)
    pltpu.sync_copy(lab_v, labo_hbm.at[pl.ds(wid * BW, BW)])

    pltpu.sync_copy(tab_hbm, tab_v.at[pl.ds(LANES, LANES)])

    lanes = lax.iota(jnp.int32, LANES)
    lanesL = lanes * L

    b0 = wid * BW
    pltpu.sync_copy(idx_hbm.at[pl.ds(b0 * L, BW * L)], idx_blk)
    mid0 = wid * (BW // 128) * 4

    def chunk_body(ci, carry):
        p = ci & 1
        l0 = ci * LC
        cbuf = coded_s.at[p]

        @pl.when(ci > 1)
        def _():
            pltpu.make_async_copy(
                cbuf, coded_hbm.at[pl.ds(0, LC), pl.ds(mid0, 16)],
                sem_c.at[p]).wait()
            pltpu.make_async_copy(
                cbuf, mask_hbm.at[pl.ds(0, LC), pl.ds(mid0, 16)],
                sem_m.at[p]).wait()

        @plsc.parallel_loop(0, LC * (BW // 128) * (128 // LANES), unroll=8)
        def _(i):
            l_i = i >> 5
            bt2 = (i >> 3) & 3
            g = i & 7
            bidx = (bt2 * 128 + g * LANES) * L + (l0 + l_i) + lanesL
            gi4 = plsc.load_gather(idx_blk, [bidx]) << 2
            for ch in range(4):
                vals = plsc.load_gather(tab_v, [gi4 + (LANES + ch)])
                cbuf[l_i, bt2 * 4 + ch, pl.ds(g * LANES, LANES)] = vals

        pltpu.async_copy(
            cbuf, coded_hbm.at[pl.ds(l0, LC), pl.ds(mid0, 16)], sem_c.at[p])
        pltpu.async_copy(
            cbuf, mask_hbm.at[pl.ds(l0, LC), pl.ds(mid0, 16)], sem_m.at[p])
        return carry
    lax.fori_loop(0, L // LC, chunk_body, 0)

    for p in range(2):
        cbuf = coded_s.at[p]
        pltpu.make_async_copy(
            cbuf, coded_hbm.at[pl.ds(0, LC), pl.ds(mid0, 16)],
            sem_c.at[p]).wait()
        pltpu.make_async_copy(
            cbuf, mask_hbm.at[pl.ds(0, LC), pl.ds(mid0, 16)],
            sem_m.at[p]).wait()


def _tc_embed_body(tab_ref, out_ref, pat_ref):
    @pl.when(pl.program_id(0) == 0)
    def _():
        mid = lax.broadcasted_iota(jnp.int32, (2048, 128), 0)
        e = (mid // 1024) * 8 + (mid % 8)
        acc = jnp.zeros((2048, 128), jnp.float32)
        for i in range(16):
            acc = jnp.where(e == i, tab_ref[i], acc)
        pat_ref[...] = acc
    pat = pat_ref[...]
    for r in range(20):
        out_ref[r] = pat


def kernel(indices, labels, table):
    B, L = indices.shape
    assert B == NW * BW
    assert L % (2 * LC) == 0
    BT = B // 128

    idx_flat = indices.astype(jnp.int32).reshape(B * L)
    tab_flat = table.reshape(16).astype(jnp.float32)

    mesh = plsc.VectorSubcoreMesh(core_axis_name="c", subcore_axis_name="s")
    body = lambda *args: _sc_body(B, L, *args)
    coded_x, mask_x, labels_o = pl.kernel(
        body,
        out_type=[
            jax.ShapeDtypeStruct((L, BT * 4, 128), jnp.float32),
            jax.ShapeDtypeStruct((L, BT * 4, 128), jnp.float32),
            jax.ShapeDtypeStruct((B,), jnp.float32),
        ],
        mesh=mesh,
        compiler_params=pltpu.CompilerParams(needs_layout_passes=False),
        scratch_types=[
            pltpu.VMEM((BW * L,), jnp.int32),
            pltpu.VMEM((2, LC, 16, 128), jnp.float32),
            pltpu.VMEM((2 * LANES,), jnp.float32),
            pltpu.VMEM((BW,), jnp.float32),
            pltpu.SemaphoreType.DMA((2,)),
            pltpu.SemaphoreType.DMA((2,)),
        ],
    )(idx_flat, tab_flat, labels)

    embed_x = pl.pallas_call(
        _tc_embed_body,
        grid=(L // 20,),
        in_specs=[pl.BlockSpec(memory_space=pltpu.SMEM)],
        out_specs=pl.BlockSpec((20, BT * 16, 128), lambda i: (i, 0, 0)),
        out_shape=jax.ShapeDtypeStruct((L, BT * 16, 128), jnp.float32),
        scratch_shapes=[pltpu.VMEM((BT * 16, 128), jnp.float32)],
    )(tab_flat)

    coded = (coded_x.reshape(L, BT, 4, 128).transpose(1, 3, 0, 2)
             .reshape(B, L, 4))
    embed = (embed_x.reshape(L, 2, BT, 8, 128).transpose(2, 4, 0, 1, 3)
             .reshape(B, L, 16))
    mask = (mask_x.reshape(L, BT, 4, 128).transpose(1, 3, 0, 2)
            .reshape(B, L, 4))
    return coded, embed, mask, labels_o
